# baseline (speedup 1.0000x reference)
import jax
import jax.numpy as jnp
from jax.experimental import pallas as pl
from jax.experimental.pallas import tpu as pltpu
from jax.experimental.pallas import tpu_sc as plsc

_CHUNK = 128


def _chunk_sel_kernel(adj_ref, table_ref, cidx_ref, scol_ref, *,
                      n_total, k_top, rn):
    a = adj_ref[...]
    n_chunks = a.shape[1] // _CHUNK
    cm = []
    for s in range(n_chunks):
        sl = a[:, s * _CHUNK:(s + 1) * _CHUNK]
        table_ref[s] = sl
        cm.append(jnp.max(sl, axis=1))
    M = jnp.stack(cm, axis=1)
    col = jax.lax.broadcasted_iota(jnp.int32, M.shape, 1)
    row = pl.program_id(0) * rn + jax.lax.broadcasted_iota(
        jnp.int32, (a.shape[0],), 0)
    for j in range(k_top):
        m = jnp.max(M, axis=1, keepdims=True)
        hit = M == m
        s = jnp.min(jnp.where(hit, col, n_chunks), axis=1)
        cidx_ref[j, :] = s * n_total + row
        scol_ref[j, :] = s * _CHUNK
        M = jnp.where(col == s[:, None], -jnp.inf, M)


def _cand_topk_kernel(cand_ref, scol_ref, gidx_ref, *, n_total, k_top,
                      n_batch):
    v = cand_ref[...]
    scol = scol_ref[...]
    colmap = scol[:, :, None] + jax.lax.broadcasted_iota(
        jnp.int32, v.shape, 2)
    for k in range(k_top):
        m = jnp.max(jnp.max(v, axis=2), axis=0)
        hit = v == m[None, :, None]
        idx = jnp.min(jnp.min(jnp.where(hit, colmap, n_total), axis=2),
                      axis=0)
        for b in range(n_batch):
            gidx_ref[b, k, :] = idx + (k * n_batch + b) * n_total
        v = jnp.where(colmap == idx[None, :, None], -jnp.inf, v)


def _matmul_kernel(x_ref, w_ref, xw_ref):
    k = pl.program_id(2)
    xw_ref[...] = jnp.dot(x_ref[...], w_ref[k],
                          preferred_element_type=jnp.float32)


def _combine_kernel(xg_ref, bs_ref, y_ref, *, k_top):
    bsv = bs_ref[...]
    acc = bsv[0, :, None] * xg_ref[0, 0, :, :]
    for k in range(1, k_top):
        acc = acc + bsv[k, :, None] * xg_ref[0, k, :, :]
    y_ref[0] = acc


def _sc_gather(table, idx_flat, cout, window):
    num_idx = idx_flat.shape[1]
    mesh = plsc.VectorSubcoreMesh(core_axis_name="core",
                                  subcore_axis_name="subcore")

    @pl.kernel(out_type=jax.ShapeDtypeStruct((num_idx, cout), table.dtype),
               mesh=mesh)
    def gather_kernel(t_hbm, i_hbm, o_hbm):
        def body(i_vmem, o_vmem):
            pltpu.sync_copy(t_hbm.at[i_vmem.at[0]], o_vmem)

        pltpu.emit_pipeline(
            body,
            grid=(num_idx // window,),
            in_specs=[pl.BlockSpec((1, window), lambda i: (0, i))],
            out_specs=[pl.BlockSpec((window, cout), lambda i: (i, 0))],
            core_axis_name=("core", "subcore"),
            dimension_semantics=(pltpu.PARALLEL,),
        )(i_hbm, o_hbm)

    return gather_kernel(table, idx_flat)


def kernel(x, adj, bs, ws):
    B, N, CIN = x.shape
    K = bs.shape[1]
    COUT = ws.shape[2]
    NCH = N // _CHUNK

    RN1 = 256
    table, cidx, scol = pl.pallas_call(
        lambda a_ref, t_ref, c_ref, s_ref: _chunk_sel_kernel(
            a_ref, t_ref, c_ref, s_ref, n_total=N, k_top=K, rn=RN1),
        grid=(N // RN1,),
        in_specs=[pl.BlockSpec((RN1, N), lambda i: (i, 0))],
        out_specs=[
            pl.BlockSpec((NCH, RN1, _CHUNK), lambda i: (0, i, 0)),
            pl.BlockSpec((K, RN1), lambda i: (0, i)),
            pl.BlockSpec((K, RN1), lambda i: (0, i)),
        ],
        out_shape=[
            jax.ShapeDtypeStruct((NCH, N, _CHUNK), jnp.float32),
            jax.ShapeDtypeStruct((K, N), jnp.int32),
            jax.ShapeDtypeStruct((K, N), jnp.int32),
        ],
    )(adj)

    cand = _sc_gather(table.reshape(NCH * N, _CHUNK),
                      cidx.reshape(1, K * N), _CHUNK, window=128)
    cand = cand.reshape(K, N, _CHUNK)

    RN3 = 256
    gidx = pl.pallas_call(
        lambda c_ref, s_ref, g_ref: _cand_topk_kernel(
            c_ref, s_ref, g_ref, n_total=N, k_top=K, n_batch=B),
        grid=(N // RN3,),
        in_specs=[
            pl.BlockSpec((K, RN3, _CHUNK), lambda i: (0, i, 0)),
            pl.BlockSpec((K, RN3), lambda i: (0, i)),
        ],
        out_specs=pl.BlockSpec((B, K, RN3), lambda i: (0, 0, i)),
        out_shape=jax.ShapeDtypeStruct((B, K, N), jnp.int32),
    )(cand, scol)

    RN_B = 1024
    NB = N // RN_B
    x2 = x.reshape(B * N, CIN).astype(jnp.bfloat16)
    ws16 = ws.astype(jnp.bfloat16)
    xw = pl.pallas_call(
        _matmul_kernel,
        grid=(B, NB, K),
        in_specs=[
            pl.BlockSpec((RN_B, CIN), lambda b, nb, k: (b * NB + nb, 0)),
            pl.BlockSpec((K, CIN, COUT), lambda b, nb, k: (0, 0, 0)),
        ],
        out_specs=pl.BlockSpec(
            (RN_B, COUT), lambda b, nb, k: (k * (B * NB) + b * NB + nb, 0)),
        out_shape=jax.ShapeDtypeStruct((K * B * N, COUT), jnp.float32),
    )(x2, ws16)

    xg = _sc_gather(xw, gidx.reshape(1, B * K * N), COUT, window=128)
    xg = xg.reshape(B, K, N, COUT)

    RN_D = 256
    bs_t = bs.T
    y = pl.pallas_call(
        lambda xg_ref, bs_ref, y_ref: _combine_kernel(
            xg_ref, bs_ref, y_ref, k_top=K),
        grid=(B, N // RN_D),
        in_specs=[
            pl.BlockSpec((1, K, RN_D, COUT), lambda b, nb: (b, 0, nb, 0)),
            pl.BlockSpec((K, RN_D), lambda b, nb: (0, nb)),
        ],
        out_specs=pl.BlockSpec((1, RN_D, COUT), lambda b, nb: (b, nb, 0)),
        out_shape=jax.ShapeDtypeStruct((B, N, COUT), jnp.float32),
    )(xg, bs_t)
    return y


# speedup vs baseline: 1.2300x; 1.2300x over previous
import jax
import jax.numpy as jnp
from jax.experimental import pallas as pl
from jax.experimental.pallas import tpu as pltpu
from jax.experimental.pallas import tpu_sc as plsc

_CHUNK = 128


def _chunk_sel_kernel(adj_ref, table_ref, cidx_ref, scol_ref, *,
                      n_total, k_top, rn):
    a = adj_ref[...]
    n_chunks = a.shape[1] // _CHUNK
    cm = []
    for s in range(n_chunks):
        sl = a[:, s * _CHUNK:(s + 1) * _CHUNK]
        table_ref[s] = sl
        cm.append(jnp.max(sl, axis=1))
    M = jnp.stack(cm, axis=1)
    col = jax.lax.broadcasted_iota(jnp.int32, M.shape, 1)
    row = pl.program_id(0) * rn + jax.lax.broadcasted_iota(
        jnp.int32, (a.shape[0],), 0)
    for j in range(k_top):
        m = jnp.max(M, axis=1, keepdims=True)
        hit = M == m
        s = jnp.min(jnp.where(hit, col, n_chunks), axis=1)
        cidx_ref[j, :] = s * n_total + row
        scol_ref[j, :] = s * _CHUNK
        M = jnp.where(col == s[:, None], -jnp.inf, M)


def _cand_topk_kernel(cand_ref, scol_ref, gidx_ref, *, n_total, k_top,
                      n_batch):
    v = cand_ref[...]
    scol = scol_ref[...]
    colmap = scol[:, :, None] + jax.lax.broadcasted_iota(
        jnp.int32, v.shape, 2)
    for k in range(k_top):
        m = jnp.max(jnp.max(v, axis=0), axis=1)
        hit = v == m[None, :, None]
        idx = jnp.min(jnp.min(jnp.where(hit, colmap, n_total), axis=0),
                      axis=1)
        for b in range(n_batch):
            gidx_ref[b, k, :] = idx + (k * n_batch + b) * n_total
        v = jnp.where(colmap == idx[None, :, None], -jnp.inf, v)


def _matmul_kernel(x_ref, w_ref, xw_ref):
    k = pl.program_id(2)
    xw_ref[...] = jnp.dot(x_ref[...], w_ref[k],
                          preferred_element_type=jnp.float32)


def _combine_kernel(xg_ref, bs_ref, y_ref, *, k_top):
    bsv = bs_ref[...]
    acc = bsv[0, :, None] * xg_ref[0, 0, :, :]
    for k in range(1, k_top):
        acc = acc + bsv[k, :, None] * xg_ref[0, k, :, :]
    y_ref[0] = acc


def _sc_gather(table, idx_flat, cout, window):
    num_idx = idx_flat.shape[1]
    mesh = plsc.VectorSubcoreMesh(core_axis_name="core",
                                  subcore_axis_name="subcore")

    @pl.kernel(out_type=jax.ShapeDtypeStruct((num_idx, cout), table.dtype),
               mesh=mesh)
    def gather_kernel(t_hbm, i_hbm, o_hbm):
        def body(i_vmem, o_vmem):
            pltpu.sync_copy(t_hbm.at[i_vmem.at[0]], o_vmem)

        pltpu.emit_pipeline(
            body,
            grid=(num_idx // window,),
            in_specs=[pl.BlockSpec((1, window), lambda i: (0, i))],
            out_specs=[pl.BlockSpec((window, cout), lambda i: (i, 0))],
            core_axis_name=("core", "subcore"),
            dimension_semantics=(pltpu.PARALLEL,),
        )(i_hbm, o_hbm)

    return gather_kernel(table, idx_flat)


def kernel(x, adj, bs, ws):
    B, N, CIN = x.shape
    K = bs.shape[1]
    COUT = ws.shape[2]
    NCH = N // _CHUNK

    RN1 = 256
    table, cidx, scol = pl.pallas_call(
        lambda a_ref, t_ref, c_ref, s_ref: _chunk_sel_kernel(
            a_ref, t_ref, c_ref, s_ref, n_total=N, k_top=K, rn=RN1),
        grid=(N // RN1,),
        in_specs=[pl.BlockSpec((RN1, N), lambda i: (i, 0))],
        out_specs=[
            pl.BlockSpec((NCH, RN1, _CHUNK), lambda i: (0, i, 0)),
            pl.BlockSpec((K, RN1), lambda i: (0, i)),
            pl.BlockSpec((K, RN1), lambda i: (0, i)),
        ],
        out_shape=[
            jax.ShapeDtypeStruct((NCH, N, _CHUNK), jnp.float32),
            jax.ShapeDtypeStruct((K, N), jnp.int32),
            jax.ShapeDtypeStruct((K, N), jnp.int32),
        ],
    )(adj)

    cand = _sc_gather(table.reshape(NCH * N, _CHUNK),
                      cidx.reshape(1, K * N), _CHUNK, window=128)
    cand = cand.reshape(K, N, _CHUNK)

    RN3 = 256
    gidx = pl.pallas_call(
        lambda c_ref, s_ref, g_ref: _cand_topk_kernel(
            c_ref, s_ref, g_ref, n_total=N, k_top=K, n_batch=B),
        grid=(N // RN3,),
        in_specs=[
            pl.BlockSpec((K, RN3, _CHUNK), lambda i: (0, i, 0)),
            pl.BlockSpec((K, RN3), lambda i: (0, i)),
        ],
        out_specs=pl.BlockSpec((B, K, RN3), lambda i: (0, 0, i)),
        out_shape=jax.ShapeDtypeStruct((B, K, N), jnp.int32),
    )(cand, scol)

    RN_B = 1024
    NB = N // RN_B
    x2 = x.reshape(B * N, CIN).astype(jnp.bfloat16)
    ws16 = ws.astype(jnp.bfloat16)
    xw = pl.pallas_call(
        _matmul_kernel,
        grid=(B, NB, K),
        in_specs=[
            pl.BlockSpec((RN_B, CIN), lambda b, nb, k: (b * NB + nb, 0)),
            pl.BlockSpec((K, CIN, COUT), lambda b, nb, k: (0, 0, 0)),
        ],
        out_specs=pl.BlockSpec(
            (RN_B, COUT), lambda b, nb, k: (k * (B * NB) + b * NB + nb, 0)),
        out_shape=jax.ShapeDtypeStruct((K * B * N, COUT), jnp.float32),
    )(x2, ws16)

    xg = _sc_gather(xw, gidx.reshape(1, B * K * N), COUT, window=128)
    xg = xg.reshape(B, K, N, COUT)

    RN_D = 256
    bs_t = bs.T
    y = pl.pallas_call(
        lambda xg_ref, bs_ref, y_ref: _combine_kernel(
            xg_ref, bs_ref, y_ref, k_top=K),
        grid=(B, N // RN_D),
        in_specs=[
            pl.BlockSpec((1, K, RN_D, COUT), lambda b, nb: (b, 0, nb, 0)),
            pl.BlockSpec((K, RN_D), lambda b, nb: (0, nb)),
        ],
        out_specs=pl.BlockSpec((1, RN_D, COUT), lambda b, nb: (b, nb, 0)),
        out_shape=jax.ShapeDtypeStruct((B, N, COUT), jnp.float32),
    )(xg, bs_t)
    return y


# speedup vs baseline: 1.3486x; 1.0964x over previous
import jax
import jax.numpy as jnp
from jax.experimental import pallas as pl
from jax.experimental.pallas import tpu as pltpu
from jax.experimental.pallas import tpu_sc as plsc

_CHUNK = 128


def _chunk_sel_kernel(adj_ref, x_ref, w_ref, table_ref, cidx_ref, scol_ref,
                      xw_ref, *, n_total, k_top, rn):
    xv = x_ref[...]
    wv = w_ref[...]
    n_batch = xv.shape[0]
    for k in range(k_top):
        for b in range(n_batch):
            xw_ref[k * n_batch + b] = jnp.dot(
                xv[b], wv[k], preferred_element_type=jnp.float32)

    a = adj_ref[...]
    n_chunks = a.shape[1] // _CHUNK
    cm = []
    for s in range(n_chunks):
        sl = a[:, s * _CHUNK:(s + 1) * _CHUNK]
        table_ref[s] = sl
        cm.append(jnp.max(sl, axis=1))
    M = jnp.stack(cm, axis=1)
    col = jax.lax.broadcasted_iota(jnp.int32, M.shape, 1)
    row = pl.program_id(0) * rn + jax.lax.broadcasted_iota(
        jnp.int32, (a.shape[0],), 0)
    for j in range(k_top):
        m = jnp.max(M, axis=1, keepdims=True)
        hit = M == m
        s = jnp.min(jnp.where(hit, col, n_chunks), axis=1)
        cidx_ref[j, :] = s * n_total + row
        scol_ref[j, :] = s * _CHUNK
        M = jnp.where(col == s[:, None], -jnp.inf, M)


def _cand_topk_kernel(cand_ref, scol_ref, gidx_ref, *, n_total, k_top,
                      n_batch):
    v = cand_ref[...]
    scol = scol_ref[...]
    colmap = scol[:, :, None] + jax.lax.broadcasted_iota(
        jnp.int32, v.shape, 2)
    for k in range(k_top):
        m = jnp.max(jnp.max(v, axis=0), axis=1)
        hit = v == m[None, :, None]
        idx = jnp.min(jnp.min(jnp.where(hit, colmap, n_total), axis=0),
                      axis=1)
        for b in range(n_batch):
            gidx_ref[b, k, :] = idx + (k * n_batch + b) * n_total
        v = jnp.where(colmap == idx[None, :, None], -jnp.inf, v)


def _combine_kernel(xg_ref, bs_ref, y_ref, *, k_top):
    bsv = bs_ref[...]
    acc = bsv[0, :, None] * xg_ref[0, 0, :, :]
    for k in range(1, k_top):
        acc = acc + bsv[k, :, None] * xg_ref[0, k, :, :]
    y_ref[0] = acc


def _sc_gather(table, idx_flat, cout, window):
    num_idx = idx_flat.shape[1]
    mesh = plsc.VectorSubcoreMesh(core_axis_name="core",
                                  subcore_axis_name="subcore")

    @pl.kernel(out_type=jax.ShapeDtypeStruct((num_idx, cout), table.dtype),
               mesh=mesh)
    def gather_kernel(t_hbm, i_hbm, o_hbm):
        def body(i_vmem, o_vmem):
            pltpu.sync_copy(t_hbm.at[i_vmem.at[0]], o_vmem)

        pltpu.emit_pipeline(
            body,
            grid=(num_idx // window,),
            in_specs=[pl.BlockSpec((1, window), lambda i: (0, i))],
            out_specs=[pl.BlockSpec((window, cout), lambda i: (i, 0))],
            core_axis_name=("core", "subcore"),
            dimension_semantics=(pltpu.PARALLEL,),
        )(i_hbm, o_hbm)

    return gather_kernel(table, idx_flat)


def kernel(x, adj, bs, ws):
    B, N, CIN = x.shape
    K = bs.shape[1]
    COUT = ws.shape[2]
    NCH = N // _CHUNK

    RN1 = 256
    x3 = x.astype(jnp.bfloat16)
    ws16 = ws.astype(jnp.bfloat16)
    table, cidx, scol, xw = pl.pallas_call(
        lambda a_ref, x_ref, w_ref, t_ref, c_ref, s_ref, o_ref:
        _chunk_sel_kernel(
            a_ref, x_ref, w_ref, t_ref, c_ref, s_ref, o_ref,
            n_total=N, k_top=K, rn=RN1),
        grid=(N // RN1,),
        in_specs=[
            pl.BlockSpec((RN1, N), lambda i: (i, 0)),
            pl.BlockSpec((B, RN1, CIN), lambda i: (0, i, 0)),
            pl.BlockSpec((K, CIN, COUT), lambda i: (0, 0, 0)),
        ],
        out_specs=[
            pl.BlockSpec((NCH, RN1, _CHUNK), lambda i: (0, i, 0)),
            pl.BlockSpec((K, RN1), lambda i: (0, i)),
            pl.BlockSpec((K, RN1), lambda i: (0, i)),
            pl.BlockSpec((K * B, RN1, COUT), lambda i: (0, i, 0)),
        ],
        out_shape=[
            jax.ShapeDtypeStruct((NCH, N, _CHUNK), jnp.float32),
            jax.ShapeDtypeStruct((K, N), jnp.int32),
            jax.ShapeDtypeStruct((K, N), jnp.int32),
            jax.ShapeDtypeStruct((K * B, N, COUT), jnp.float32),
        ],
    )(adj, x3, ws16)
    xw = xw.reshape(K * B * N, COUT)

    cand = _sc_gather(table.reshape(NCH * N, _CHUNK),
                      cidx.reshape(1, K * N), _CHUNK, window=128)
    cand = cand.reshape(K, N, _CHUNK)

    RN3 = 256
    gidx = pl.pallas_call(
        lambda c_ref, s_ref, g_ref: _cand_topk_kernel(
            c_ref, s_ref, g_ref, n_total=N, k_top=K, n_batch=B),
        grid=(N // RN3,),
        in_specs=[
            pl.BlockSpec((K, RN3, _CHUNK), lambda i: (0, i, 0)),
            pl.BlockSpec((K, RN3), lambda i: (0, i)),
        ],
        out_specs=pl.BlockSpec((B, K, RN3), lambda i: (0, 0, i)),
        out_shape=jax.ShapeDtypeStruct((B, K, N), jnp.int32),
    )(cand, scol)

    xg = _sc_gather(xw, gidx.reshape(1, B * K * N), COUT, window=128)
    xg = xg.reshape(B, K, N, COUT)

    RN_D = 256
    bs_t = bs.T
    y = pl.pallas_call(
        lambda xg_ref, bs_ref, y_ref: _combine_kernel(
            xg_ref, bs_ref, y_ref, k_top=K),
        grid=(B, N // RN_D),
        in_specs=[
            pl.BlockSpec((1, K, RN_D, COUT), lambda b, nb: (b, 0, nb, 0)),
            pl.BlockSpec((K, RN_D), lambda b, nb: (0, nb)),
        ],
        out_specs=pl.BlockSpec((1, RN_D, COUT), lambda b, nb: (b, nb, 0)),
        out_shape=jax.ShapeDtypeStruct((B, N, COUT), jnp.float32),
    )(xg, bs_t)
    return y


# speedup vs baseline: 1.3912x; 1.0316x over previous
import jax
import jax.numpy as jnp
from jax.experimental import pallas as pl
from jax.experimental.pallas import tpu as pltpu
from jax.experimental.pallas import tpu_sc as plsc

_CHUNK = 128


def _chunk_sel_kernel(adj_ref, x_ref, w_ref, table_ref, cidx_ref, scol_ref,
                      xw_ref, *, n_total, k_top, rn):
    xv = x_ref[...].astype(jnp.bfloat16)
    wv = w_ref[...]
    n_batch = xv.shape[0]
    for k in range(k_top):
        for b in range(n_batch):
            xw_ref[k * n_batch + b] = jnp.dot(
                xv[b], wv[k], preferred_element_type=jnp.float32)

    a = adj_ref[...]
    n_chunks = a.shape[1] // _CHUNK
    cm = []
    for s in range(n_chunks):
        sl = a[:, s * _CHUNK:(s + 1) * _CHUNK]
        table_ref[s] = sl
        cm.append(jnp.max(sl, axis=1))
    M = jnp.stack(cm, axis=1)
    col = jax.lax.broadcasted_iota(jnp.int32, M.shape, 1)
    row = pl.program_id(0) * rn + jax.lax.broadcasted_iota(
        jnp.int32, (a.shape[0],), 0)
    for j in range(k_top):
        m = jnp.max(M, axis=1, keepdims=True)
        hit = M == m
        s = jnp.min(jnp.where(hit, col, n_chunks), axis=1)
        cidx_ref[j, :] = s * n_total + row
        scol_ref[j, :] = s * _CHUNK
        M = jnp.where(col == s[:, None], -jnp.inf, M)


def _cand_topk_kernel(cand_ref, scol_ref, gidx_ref, *, n_total, k_top,
                      n_batch):
    v = cand_ref[...]
    scol = scol_ref[...]
    colmap = scol[:, :, None] + jax.lax.broadcasted_iota(
        jnp.int32, v.shape, 2)
    for k in range(k_top):
        m = jnp.max(jnp.max(v, axis=0), axis=1)
        hit = v == m[None, :, None]
        idx = jnp.min(jnp.min(jnp.where(hit, colmap, n_total), axis=0),
                      axis=1)
        for b in range(n_batch):
            gidx_ref[b, k, :] = idx + (k * n_batch + b) * n_total
        v = jnp.where(colmap == idx[None, :, None], -jnp.inf, v)


def _combine_kernel(xg_ref, bs_ref, y_ref, *, k_top):
    bsv = bs_ref[...]
    acc = bsv[:, 0:1] * xg_ref[0, 0, :, :]
    for k in range(1, k_top):
        acc = acc + bsv[:, k:k + 1] * xg_ref[0, k, :, :]
    y_ref[0] = acc


def _sc_gather(table, idx, cout, window):
    num_idx = 1
    for d in idx.shape:
        num_idx *= d
    nw = idx.shape[-1] // window
    if idx.ndim == 2:
        idx_spec = pl.BlockSpec((1, window), lambda i: (i // nw, i % nw))
    else:
        mid = idx.shape[1]
        idx_spec = pl.BlockSpec(
            (1, 1, window),
            lambda i: (i // (mid * nw), (i // nw) % mid, i % nw))
    mesh = plsc.VectorSubcoreMesh(core_axis_name="core",
                                  subcore_axis_name="subcore")

    @pl.kernel(out_type=jax.ShapeDtypeStruct((num_idx, cout), table.dtype),
               mesh=mesh)
    def gather_kernel(t_hbm, i_hbm, o_hbm):
        def body(i_vmem, o_vmem):
            iv = i_vmem.at[0] if i_vmem.ndim == 2 else i_vmem.at[0, 0]
            pltpu.sync_copy(t_hbm.at[iv], o_vmem)

        pltpu.emit_pipeline(
            body,
            grid=(num_idx // window,),
            in_specs=[idx_spec],
            out_specs=[pl.BlockSpec((window, cout), lambda i: (i, 0))],
            core_axis_name=("core", "subcore"),
            dimension_semantics=(pltpu.PARALLEL,),
        )(i_hbm, o_hbm)

    return gather_kernel(table, idx)


def kernel(x, adj, bs, ws):
    B, N, CIN = x.shape
    K = bs.shape[1]
    COUT = ws.shape[2]
    NCH = N // _CHUNK

    RN1 = 256
    ws16 = ws.astype(jnp.bfloat16)
    table, cidx, scol, xw = pl.pallas_call(
        lambda a_ref, x_ref, w_ref, t_ref, c_ref, s_ref, o_ref:
        _chunk_sel_kernel(
            a_ref, x_ref, w_ref, t_ref, c_ref, s_ref, o_ref,
            n_total=N, k_top=K, rn=RN1),
        grid=(N // RN1,),
        in_specs=[
            pl.BlockSpec((RN1, N), lambda i: (i, 0)),
            pl.BlockSpec((B, RN1, CIN), lambda i: (0, i, 0)),
            pl.BlockSpec((K, CIN, COUT), lambda i: (0, 0, 0)),
        ],
        out_specs=[
            pl.BlockSpec((NCH, RN1, _CHUNK), lambda i: (0, i, 0)),
            pl.BlockSpec((K, RN1), lambda i: (0, i)),
            pl.BlockSpec((K, RN1), lambda i: (0, i)),
            pl.BlockSpec((K * B, RN1, COUT), lambda i: (0, i, 0)),
        ],
        out_shape=[
            jax.ShapeDtypeStruct((NCH, N, _CHUNK), jnp.float32),
            jax.ShapeDtypeStruct((K, N), jnp.int32),
            jax.ShapeDtypeStruct((K, N), jnp.int32),
            jax.ShapeDtypeStruct((K * B, N, COUT), jnp.float32),
        ],
    )(adj, x, ws16)
    xw = xw.reshape(K * B * N, COUT)

    cand = _sc_gather(table.reshape(NCH * N, _CHUNK), cidx, _CHUNK,
                      window=128)
    cand = cand.reshape(K, N, _CHUNK)

    RN3 = 256
    gidx = pl.pallas_call(
        lambda c_ref, s_ref, g_ref: _cand_topk_kernel(
            c_ref, s_ref, g_ref, n_total=N, k_top=K, n_batch=B),
        grid=(N // RN3,),
        in_specs=[
            pl.BlockSpec((K, RN3, _CHUNK), lambda i: (0, i, 0)),
            pl.BlockSpec((K, RN3), lambda i: (0, i)),
        ],
        out_specs=pl.BlockSpec((B, K, RN3), lambda i: (0, 0, i)),
        out_shape=jax.ShapeDtypeStruct((B, K, N), jnp.int32),
    )(cand, scol)

    xg = _sc_gather(xw, gidx, COUT, window=128)
    xg = xg.reshape(B, K, N, COUT)

    RN_D = 256
    y = pl.pallas_call(
        lambda xg_ref, bs_ref, y_ref: _combine_kernel(
            xg_ref, bs_ref, y_ref, k_top=K),
        grid=(B, N // RN_D),
        in_specs=[
            pl.BlockSpec((1, K, RN_D, COUT), lambda b, nb: (b, 0, nb, 0)),
            pl.BlockSpec((RN_D, K), lambda b, nb: (nb, 0)),
        ],
        out_specs=pl.BlockSpec((1, RN_D, COUT), lambda b, nb: (b, nb, 0)),
        out_shape=jax.ShapeDtypeStruct((B, N, COUT), jnp.float32),
    )(xg, bs)
    return y


# speedup vs baseline: 1.3980x; 1.0049x over previous
import jax
import jax.numpy as jnp
from jax.experimental import pallas as pl
from jax.experimental.pallas import tpu as pltpu
from jax.experimental.pallas import tpu_sc as plsc


def _topk_mm_kernel(adj_ref, x_ref, w_ref, gidx_ref, xw_ref, *,
                    n_total, k_top):
    xv = x_ref[...].astype(jnp.bfloat16)
    wv = w_ref[...]
    n_batch = xv.shape[0]
    for k in range(k_top):
        for b in range(n_batch):
            xw_ref[k * n_batch + b] = jnp.dot(
                xv[b], wv[k], preferred_element_type=jnp.float32)

    a = adj_ref[...]
    col = jax.lax.broadcasted_iota(jnp.int32, a.shape, 1)
    for k in range(k_top):
        m = jnp.max(a, axis=1, keepdims=True)
        hit = a == m
        idx = jnp.min(jnp.where(hit, col, n_total), axis=1)
        for b in range(n_batch):
            gidx_ref[b, k, :] = idx + (k * n_batch + b) * n_total
        a = jnp.where(col == idx[:, None], -jnp.inf, a)


def _combine_kernel(xg_ref, bs_ref, y_ref, *, k_top):
    bsv = bs_ref[...]
    acc = bsv[:, 0:1] * xg_ref[0, 0, :, :]
    for k in range(1, k_top):
        acc = acc + bsv[:, k:k + 1] * xg_ref[0, k, :, :]
    y_ref[0] = acc


def _sc_gather(table, idx, cout, window):
    num_idx = 1
    for d in idx.shape:
        num_idx *= d
    nw = idx.shape[-1] // window
    if idx.ndim == 2:
        idx_spec = pl.BlockSpec((1, window), lambda i: (i // nw, i % nw))
    else:
        mid = idx.shape[1]
        idx_spec = pl.BlockSpec(
            (1, 1, window),
            lambda i: (i // (mid * nw), (i // nw) % mid, i % nw))
    mesh = plsc.VectorSubcoreMesh(core_axis_name="core",
                                  subcore_axis_name="subcore")

    @pl.kernel(out_type=jax.ShapeDtypeStruct((num_idx, cout), table.dtype),
               mesh=mesh)
    def gather_kernel(t_hbm, i_hbm, o_hbm):
        def body(i_vmem, o_vmem):
            iv = i_vmem.at[0] if i_vmem.ndim == 2 else i_vmem.at[0, 0]
            pltpu.sync_copy(t_hbm.at[iv], o_vmem)

        pltpu.emit_pipeline(
            body,
            grid=(num_idx // window,),
            in_specs=[idx_spec],
            out_specs=[pl.BlockSpec((window, cout), lambda i: (i, 0))],
            core_axis_name=("core", "subcore"),
            dimension_semantics=(pltpu.PARALLEL,),
        )(i_hbm, o_hbm)

    return gather_kernel(table, idx)


def kernel(x, adj, bs, ws):
    B, N, CIN = x.shape
    K = bs.shape[1]
    COUT = ws.shape[2]

    RN1 = 256
    ws16 = ws.astype(jnp.bfloat16)
    gidx, xw = pl.pallas_call(
        lambda a_ref, x_ref, w_ref, g_ref, o_ref: _topk_mm_kernel(
            a_ref, x_ref, w_ref, g_ref, o_ref, n_total=N, k_top=K),
        grid=(N // RN1,),
        in_specs=[
            pl.BlockSpec((RN1, N), lambda i: (i, 0)),
            pl.BlockSpec((B, RN1, CIN), lambda i: (0, i, 0)),
            pl.BlockSpec((K, CIN, COUT), lambda i: (0, 0, 0)),
        ],
        out_specs=[
            pl.BlockSpec((B, K, RN1), lambda i: (0, 0, i)),
            pl.BlockSpec((K * B, RN1, COUT), lambda i: (0, i, 0)),
        ],
        out_shape=[
            jax.ShapeDtypeStruct((B, K, N), jnp.int32),
            jax.ShapeDtypeStruct((K * B, N, COUT), jnp.float32),
        ],
    )(adj, x, ws16)

    xg = _sc_gather(xw.reshape(K * B * N, COUT), gidx, COUT, window=128)
    xg = xg.reshape(B, K, N, COUT)

    RN_D = 256
    y = pl.pallas_call(
        lambda xg_ref, bs_ref, y_ref: _combine_kernel(
            xg_ref, bs_ref, y_ref, k_top=K),
        grid=(B, N // RN_D),
        in_specs=[
            pl.BlockSpec((1, K, RN_D, COUT), lambda b, nb: (b, 0, nb, 0)),
            pl.BlockSpec((RN_D, K), lambda b, nb: (nb, 0)),
        ],
        out_specs=pl.BlockSpec((1, RN_D, COUT), lambda b, nb: (b, nb, 0)),
        out_shape=jax.ShapeDtypeStruct((B, N, COUT), jnp.float32),
    )(xg, bs)
    return y


# speedup vs baseline: 1.4178x; 1.0142x over previous
import jax
import jax.numpy as jnp
from jax.experimental import pallas as pl
from jax.experimental.pallas import tpu as pltpu
from jax.experimental.pallas import tpu_sc as plsc


def _topk_mm_kernel(adj_ref, x_ref, w_ref, gidx_ref, xw_ref, *,
                    n_total, k_top):
    xv = x_ref[...].astype(jnp.bfloat16)
    wv = w_ref[...]
    n_batch = xv.shape[0]
    for k in range(k_top):
        for b in range(n_batch):
            xw_ref[k * n_batch + b] = jnp.dot(
                xv[b], wv[k], preferred_element_type=jnp.float32)

    a = adj_ref[...]
    col = jax.lax.broadcasted_iota(jnp.int32, a.shape, 1)
    for k in range(k_top):
        m = jnp.max(a, axis=1, keepdims=True)
        hit = a == m
        idx = jnp.min(jnp.where(hit, col, n_total), axis=1)
        for b in range(n_batch):
            gidx_ref[b, k, :] = idx + (k * n_batch + b) * n_total
        a = jnp.where(col == idx[:, None], -jnp.inf, a)


def _combine_kernel(xg_ref, bs_ref, y_ref, *, k_top):
    bsv = bs_ref[...]
    acc = bsv[:, 0:1] * xg_ref[0, 0, :, :]
    for k in range(1, k_top):
        acc = acc + bsv[:, k:k + 1] * xg_ref[0, k, :, :]
    y_ref[0] = acc


def _sc_gather(table, idx, cout, window):
    num_idx = 1
    for d in idx.shape:
        num_idx *= d
    nw = idx.shape[-1] // window
    if idx.ndim == 2:
        idx_spec = pl.BlockSpec((1, window), lambda i: (i // nw, i % nw))
    else:
        mid = idx.shape[1]
        idx_spec = pl.BlockSpec(
            (1, 1, window),
            lambda i: (i // (mid * nw), (i // nw) % mid, i % nw))
    mesh = plsc.VectorSubcoreMesh(core_axis_name="core",
                                  subcore_axis_name="subcore")

    @pl.kernel(out_type=jax.ShapeDtypeStruct((num_idx, cout), table.dtype),
               mesh=mesh)
    def gather_kernel(t_hbm, i_hbm, o_hbm):
        def body(i_vmem, o_vmem):
            iv = i_vmem.at[0] if i_vmem.ndim == 2 else i_vmem.at[0, 0]
            pltpu.sync_copy(t_hbm.at[iv], o_vmem)

        pltpu.emit_pipeline(
            body,
            grid=(num_idx // window,),
            in_specs=[idx_spec],
            out_specs=[pl.BlockSpec((window, cout), lambda i: (i, 0))],
            core_axis_name=("core", "subcore"),
            dimension_semantics=(pltpu.PARALLEL,),
        )(i_hbm, o_hbm)

    return gather_kernel(table, idx)


def kernel(x, adj, bs, ws):
    B, N, CIN = x.shape
    K = bs.shape[1]
    COUT = ws.shape[2]

    RN1 = 512
    ws16 = ws.astype(jnp.bfloat16)
    gidx, xw = pl.pallas_call(
        lambda a_ref, x_ref, w_ref, g_ref, o_ref: _topk_mm_kernel(
            a_ref, x_ref, w_ref, g_ref, o_ref, n_total=N, k_top=K),
        grid=(N // RN1,),
        in_specs=[
            pl.BlockSpec((RN1, N), lambda i: (i, 0)),
            pl.BlockSpec((B, RN1, CIN), lambda i: (0, i, 0)),
            pl.BlockSpec((K, CIN, COUT), lambda i: (0, 0, 0)),
        ],
        out_specs=[
            pl.BlockSpec((B, K, RN1), lambda i: (0, 0, i)),
            pl.BlockSpec((K * B, RN1, COUT), lambda i: (0, i, 0)),
        ],
        out_shape=[
            jax.ShapeDtypeStruct((B, K, N), jnp.int32),
            jax.ShapeDtypeStruct((K * B, N, COUT), jnp.float32),
        ],
    )(adj, x, ws16)

    xg = _sc_gather(xw.reshape(K * B * N, COUT), gidx, COUT, window=128)
    xg = xg.reshape(B, K, N, COUT)

    RN_D = 256
    y = pl.pallas_call(
        lambda xg_ref, bs_ref, y_ref: _combine_kernel(
            xg_ref, bs_ref, y_ref, k_top=K),
        grid=(B, N // RN_D),
        in_specs=[
            pl.BlockSpec((1, K, RN_D, COUT), lambda b, nb: (b, 0, nb, 0)),
            pl.BlockSpec((RN_D, K), lambda b, nb: (nb, 0)),
        ],
        out_specs=pl.BlockSpec((1, RN_D, COUT), lambda b, nb: (b, nb, 0)),
        out_shape=jax.ShapeDtypeStruct((B, N, COUT), jnp.float32),
    )(xg, bs)
    return y
